# baseline (speedup 1.0000x reference)
import jax
import jax.numpy as jnp
from jax import lax
from jax.experimental import pallas as pl

IOU_THR = 0.5


def _make_nms_body(B: int, nb: int):
    def body(rows_ref, cols_ref, keep_ref):
        def process_block(bi, _):
            base = bi * B
            x1c = jnp.broadcast_to(cols_ref[pl.ds(base, B), 0:1], (B, B))
            y1c = jnp.broadcast_to(cols_ref[pl.ds(base, B), 1:2], (B, B))
            x2c = jnp.broadcast_to(cols_ref[pl.ds(base, B), 2:3], (B, B))
            y2c = jnp.broadcast_to(cols_ref[pl.ds(base, B), 3:4], (B, B))
            areac = (x2c - x1c) * (y2c - y1c)

            def over_tile(bj_start):
                x1r = rows_ref[0:1, pl.ds(bj_start, B)]
                y1r = rows_ref[1:2, pl.ds(bj_start, B)]
                x2r = rows_ref[2:3, pl.ds(bj_start, B)]
                y2r = rows_ref[3:4, pl.ds(bj_start, B)]
                arear = (x2r - x1r) * (y2r - y1r)
                w = jnp.maximum(
                    jnp.minimum(x2c, x2r) - jnp.maximum(x1c, x1r), 0.0)
                h = jnp.maximum(
                    jnp.minimum(y2c, y2r) - jnp.maximum(y1c, y1r), 0.0)
                inter = w * h
                union = areac + arear - inter
                iou = inter / (union + 1e-8)
                return (iou > IOU_THR).astype(jnp.float32)

            def cross(bj, supp):
                over = over_tile(bj * B)
                kprev = keep_ref[pl.ds(bj * B, B), 0:1]
                return supp + lax.dot_general(
                    over, kprev, (((1,), (0,)), ((), ())),
                    preferred_element_type=jnp.float32)

            supp = lax.fori_loop(0, bi, cross, jnp.zeros((B, 1), jnp.float32))
            active = (supp < 0.5).astype(jnp.float32)

            over_d = over_tile(base)
            ri = lax.broadcasted_iota(jnp.int32, (B, B), 0)
            ci = lax.broadcasted_iota(jnp.int32, (B, B), 1)
            tri = over_d * (ci < ri).astype(jnp.float32)

            def cond(c):
                it, _, changed = c
                return jnp.logical_and(changed, it < B)

            def sweep(c):
                it, keep, _ = c
                s = lax.dot_general(
                    tri, keep, (((1,), (0,)), ((), ())),
                    preferred_element_type=jnp.float32)
                new = jnp.where(s > 0.5, 0.0, active)
                return it + 1, new, jnp.any(new != keep)

            _, keep_blk, _ = lax.while_loop(
                cond, sweep, (jnp.int32(0), active, jnp.bool_(True)))
            keep_ref[pl.ds(base, B), 0:1] = keep_blk
            return 0

        lax.fori_loop(0, nb, process_block, 0)

    return body


@jax.jit
def kernel(boxes, scores):
    n = boxes.shape[0]
    B = 1024
    nb = -(-n // B)
    npad = nb * B

    scores_p = jnp.concatenate(
        [scores, jnp.full((npad - n,), -1.0, scores.dtype)])
    boxes_p = jnp.concatenate(
        [boxes, jnp.zeros((npad - n, 4), boxes.dtype)])

    _, x1s, y1s, x2s, y2s, order = lax.sort(
        (-scores_p, boxes_p[:, 0], boxes_p[:, 1], boxes_p[:, 2],
         boxes_p[:, 3], jnp.arange(npad, dtype=jnp.int32)),
        num_keys=1)

    rows = jnp.zeros((8, npad), jnp.float32)
    rows = rows.at[0].set(x1s).at[1].set(y1s).at[2].set(x2s).at[3].set(y2s)
    sboxes = jnp.stack([x1s, y1s, x2s, y2s], axis=1)

    keep_s = pl.pallas_call(
        _make_nms_body(B, nb),
        out_shape=jax.ShapeDtypeStruct((npad, 1), jnp.float32),
    )(rows, sboxes)

    keep_sorted = keep_s[:, 0] > 0.5
    keep = jnp.zeros((npad,), bool).at[order].set(keep_sorted)[:n]
    kept_scores = scores * keep.astype(scores.dtype)
    return keep, kept_scores


# speedup vs baseline: 1.1848x; 1.1848x over previous
import functools

import jax
import jax.numpy as jnp
from jax import lax
from jax.experimental import pallas as pl
from jax.experimental.pallas import tpu as pltpu
from jax.experimental.pallas import tpu_sc as plsc

IOU_THR = 0.5


def _make_nms_body(B: int, nb: int):
    def body(rows_ref, cols_ref, keep_ref):
        def process_block(bi, _):
            base = bi * B
            x1c = jnp.broadcast_to(cols_ref[pl.ds(base, B), 0:1], (B, B))
            y1c = jnp.broadcast_to(cols_ref[pl.ds(base, B), 1:2], (B, B))
            x2c = jnp.broadcast_to(cols_ref[pl.ds(base, B), 2:3], (B, B))
            y2c = jnp.broadcast_to(cols_ref[pl.ds(base, B), 3:4], (B, B))
            areac = (x2c - x1c) * (y2c - y1c)

            def over_tile(bj_start):
                x1r = rows_ref[0:1, pl.ds(bj_start, B)]
                y1r = rows_ref[1:2, pl.ds(bj_start, B)]
                x2r = rows_ref[2:3, pl.ds(bj_start, B)]
                y2r = rows_ref[3:4, pl.ds(bj_start, B)]
                arear = (x2r - x1r) * (y2r - y1r)
                w = jnp.maximum(
                    jnp.minimum(x2c, x2r) - jnp.maximum(x1c, x1r), 0.0)
                h = jnp.maximum(
                    jnp.minimum(y2c, y2r) - jnp.maximum(y1c, y1r), 0.0)
                inter = w * h
                union = areac + arear - inter
                iou = inter / (union + 1e-8)
                return (iou > IOU_THR).astype(jnp.float32)

            def cross(bj, supp):
                over = over_tile(bj * B)
                kprev = keep_ref[pl.ds(bj * B, B), 0:1]
                return supp + lax.dot_general(
                    over, kprev, (((1,), (0,)), ((), ())),
                    preferred_element_type=jnp.float32)

            supp = lax.fori_loop(0, bi, cross, jnp.zeros((B, 1), jnp.float32))
            active = (supp < 0.5).astype(jnp.float32)

            over_d = over_tile(base)
            ri = lax.broadcasted_iota(jnp.int32, (B, B), 0)
            ci = lax.broadcasted_iota(jnp.int32, (B, B), 1)
            tri = over_d * (ci < ri).astype(jnp.float32)

            def cond(c):
                it, _, changed = c
                return jnp.logical_and(changed, it < B)

            def sweep(c):
                it, keep, _ = c
                s = lax.dot_general(
                    tri, keep, (((1,), (0,)), ((), ())),
                    preferred_element_type=jnp.float32)
                new = jnp.where(s > 0.5, 0.0, active)
                return it + 1, new, jnp.any(new != keep)

            _, keep_blk, _ = lax.while_loop(
                cond, sweep, (jnp.int32(0), active, jnp.bool_(True)))
            keep_ref[pl.ds(base, B), 0:1] = keep_blk
            return 0

        lax.fori_loop(0, nb, process_block, 0)

    return body


def _make_sc_scatter(npad: int):
    nsub = 16
    chunk = npad // nsub
    R = chunk // 128
    mesh = plsc.VectorSubcoreMesh(
        core_axis_name="c", subcore_axis_name="s", num_cores=2)

    @functools.partial(
        pl.kernel,
        mesh=mesh,
        out_type=[
            jax.ShapeDtypeStruct((npad,), jnp.float32),
            jax.ShapeDtypeStruct((npad,), jnp.float32),
        ],
        scratch_types=[
            pltpu.VMEM((R, 128), jnp.int32),
            pltpu.VMEM((R, 128), jnp.float32),
            pltpu.VMEM_SHARED((npad,), jnp.float32),
            pltpu.VMEM((chunk,), jnp.float32),
            pltpu.VMEM((chunk,), jnp.float32),
            pltpu.VMEM((chunk,), jnp.float32),
        ],
    )
    def scat(order_hbm, keeps_hbm, scores_hbm, keep_out, ks_out,
             idx_v, val_v, shared, kc_v, sc_v, prod_v):
        cid = lax.axis_index("c")
        sid = lax.axis_index("s")

        @pl.when(cid == 0)
        def _():
            base = sid * chunk
            pltpu.sync_copy(order_hbm.at[sid], idx_v)
            pltpu.sync_copy(keeps_hbm.at[sid], val_v)
            for j in range(R):
                pltpu.sync_copy(val_v.at[j], shared.at[idx_v.at[j]])
            plsc.subcore_barrier()
            pltpu.sync_copy(shared.at[pl.ds(base, chunk)], kc_v)
            pltpu.sync_copy(scores_hbm.at[pl.ds(base, chunk)], sc_v)

            def mul(i, _):
                o = i * 16
                prod_v[pl.ds(o, 16)] = kc_v[pl.ds(o, 16)] * sc_v[pl.ds(o, 16)]
                return 0

            lax.fori_loop(0, chunk // 16, mul, 0)
            pltpu.sync_copy(kc_v, keep_out.at[pl.ds(base, chunk)])
            pltpu.sync_copy(prod_v, ks_out.at[pl.ds(base, chunk)])

    return scat


@jax.jit
def kernel(boxes, scores):
    n = boxes.shape[0]
    B = 1024
    nb = -(-n // B)
    npad = nb * B

    scores_p = jnp.concatenate(
        [scores, jnp.full((npad - n,), -1.0, scores.dtype)])
    boxes_p = jnp.concatenate(
        [boxes, jnp.zeros((npad - n, 4), boxes.dtype)])

    _, x1s, y1s, x2s, y2s, order = lax.sort(
        (-scores_p, boxes_p[:, 0], boxes_p[:, 1], boxes_p[:, 2],
         boxes_p[:, 3], jnp.arange(npad, dtype=jnp.int32)),
        num_keys=1)

    rows = jnp.zeros((8, npad), jnp.float32)
    rows = rows.at[0].set(x1s).at[1].set(y1s).at[2].set(x2s).at[3].set(y2s)
    sboxes = jnp.stack([x1s, y1s, x2s, y2s], axis=1)

    keep_s = pl.pallas_call(
        _make_nms_body(B, nb),
        out_shape=jax.ShapeDtypeStruct((npad, 1), jnp.float32),
    )(rows, sboxes)

    scat = _make_sc_scatter(npad)
    keep_f, ks_f = scat(
        order.reshape(16, -1, 128),
        keep_s.reshape(16, -1, 128),
        scores_p)
    keep = keep_f[:n] > 0.5
    kept_scores = ks_f[:n]
    return keep, kept_scores
